# baseline (speedup 1.0000x reference)
import jax
import jax.numpy as jnp
from jax.experimental import pallas as pl

TOKENS = 32768
D = 1024
E = 8
BT = 2048


def _gating_kernel(x_ref, wt_ref, b_ref, out_ref):
    out_ref[...] = (
        jnp.dot(x_ref[...], wt_ref[...], preferred_element_type=jnp.float32)
        + b_ref[...]
    )


def kernel(x, W, b, train):
    wt = W.T
    b2 = b.reshape(1, E)
    gates = pl.pallas_call(
        _gating_kernel,
        grid=(TOKENS // BT,),
        in_specs=[
            pl.BlockSpec((BT, D), lambda i: (i, 0)),
            pl.BlockSpec((D, E), lambda i: (0, 0)),
            pl.BlockSpec((1, E), lambda i: (0, 0)),
        ],
        out_specs=pl.BlockSpec((BT, E), lambda i: (i, 0)),
        out_shape=jax.ShapeDtypeStruct((TOKENS, E), jnp.float32),
    )(x, wt, b2)
    return (gates, gates)


# speedup vs baseline: 1.0038x; 1.0038x over previous
import jax
import jax.numpy as jnp
from jax.experimental import pallas as pl
from jax.experimental.pallas import tpu as pltpu

TOKENS = 32768
D = 1024
E = 8
BT = 4096


def _gating_kernel(x_ref, wt_ref, b_ref, out_ref):
    out_ref[...] = (
        jnp.dot(x_ref[...], wt_ref[...], preferred_element_type=jnp.float32)
        + b_ref[...]
    )


def kernel(x, W, b, train):
    wt = W.T
    b2 = b.reshape(1, E)
    gates = pl.pallas_call(
        _gating_kernel,
        grid=(TOKENS // BT,),
        in_specs=[
            pl.BlockSpec((BT, D), lambda i: (i, 0)),
            pl.BlockSpec((D, E), lambda i: (0, 0)),
            pl.BlockSpec((1, E), lambda i: (0, 0)),
        ],
        out_specs=pl.BlockSpec((BT, E), lambda i: (i, 0)),
        out_shape=jax.ShapeDtypeStruct((TOKENS, E), jnp.float32),
        compiler_params=pltpu.CompilerParams(
            dimension_semantics=("parallel",),
        ),
    )(x, wt, b2)
    return (gates, gates)


# speedup vs baseline: 1.0071x; 1.0033x over previous
import jax
import jax.numpy as jnp
from jax.experimental import pallas as pl
from jax.experimental.pallas import tpu as pltpu

TOKENS = 32768
D = 1024
E = 8
BT = 2048
GRID = TOKENS // (2 * BT)


def _gating_kernel(xa_ref, xb_ref, wt_ref, b_ref, out_ref):
    wt = wt_ref[...]
    b = b_ref[...]
    out_ref[:BT, :] = (
        jnp.dot(xa_ref[...], wt, preferred_element_type=jnp.float32) + b
    )
    out_ref[BT:, :] = (
        jnp.dot(xb_ref[...], wt, preferred_element_type=jnp.float32) + b
    )


def kernel(x, W, b, train):
    wt = W.T
    b2 = b.reshape(1, E)
    gates = pl.pallas_call(
        _gating_kernel,
        grid=(GRID,),
        in_specs=[
            pl.BlockSpec((BT, D), lambda i: (2 * i, 0)),
            pl.BlockSpec((BT, D), lambda i: (2 * i + 1, 0)),
            pl.BlockSpec((D, E), lambda i: (0, 0)),
            pl.BlockSpec((1, E), lambda i: (0, 0)),
        ],
        out_specs=pl.BlockSpec((2 * BT, E), lambda i: (i, 0)),
        out_shape=jax.ShapeDtypeStruct((TOKENS, E), jnp.float32),
        compiler_params=pltpu.CompilerParams(
            dimension_semantics=("parallel",),
        ),
    )(x, x, wt, b2)
    return (gates, gates)
